# baseline (speedup 1.0000x reference)
import functools

import jax
import jax.numpy as jnp
from jax import lax
from jax.experimental import pallas as pl
from jax.experimental.pallas import tpu as pltpu
from jax.experimental.pallas import tpu_sc as plsc

N_NODES = 10000
N_EDGES = 320000
D_FEAT = 128
D_HID = 16
N_CLASSES = 10

NC = 2
NS = 16
NW = NC * NS
CHUNK = 128
NPAD = 10240
NPT = NPAD // NS
CPT = 80
NBUF = 6
EPAD = NW * CPT * CHUNK


def _mesh():
    return plsc.VectorSubcoreMesh(
        core_axis_name="c", subcore_axis_name="s", num_cores=NC, num_subcores=NS
    )


RCHUNKS = N_EDGES // CHUNK


@functools.partial(
    pl.kernel,
    out_type=jax.ShapeDtypeStruct((NC * NPAD,), jnp.float32),
    mesh=_mesh(),
    scratch_types=[
        pltpu.VMEM((NBUF, 2, CHUNK), jnp.int32),
        pltpu.VMEM((CHUNK,), jnp.float32),
        pltpu.VMEM((NPAD,), jnp.float32),
        pltpu.VMEM_SHARED((NPAD,), jnp.float32),
        pltpu.SemaphoreType.DMA((NBUF,)),
        pltpu.SemaphoreType.DMA((NBUF,)),
    ],
)
def _deg_kernel(ei_hbm, ones_hbm, zero_hbm, out_hbm, pair_v, ones_v, stage_v,
                acc_sh, gsem, ssem):
    cid = lax.axis_index("c")
    sid = lax.axis_index("s")
    wid = sid * NC + cid
    pltpu.sync_copy(ones_hbm, ones_v)

    @pl.when(sid == 0)
    def _():
        pltpu.sync_copy(zero_hbm, stage_v)
        pltpu.sync_copy(stage_v, acc_sh)

    plsc.subcore_barrier()

    base = 78 * wid + jnp.minimum(wid, 4)
    n_w = jnp.where(wid < 4, 79, 78)

    for k in range(NBUF - 1):
        @pl.when(k < n_w)
        def _():
            pltpu.async_copy(ei_hbm.at[0:2, pl.ds((base + k) * CHUNK, CHUNK)],
                             pair_v.at[k], gsem.at[k])

    def body(j, carry):
        b = lax.rem(j, NBUF)
        pltpu.make_async_copy(ei_hbm.at[0:2, pl.ds((base + j) * CHUNK, CHUNK)],
                              pair_v.at[b], gsem.at[b]).wait()
        pltpu.async_copy(ones_v, acc_sh.at[pair_v.at[b, 1]], ssem.at[b],
                         add=True)
        jn = j + NBUF - 1

        @pl.when(jn < n_w)
        def _():
            bn = lax.rem(jn, NBUF)

            @pl.when(j >= 1)
            def _():
                pltpu.make_async_copy(ones_v, acc_sh.at[pair_v.at[bn, 1]],
                                      ssem.at[bn]).wait()

            pltpu.async_copy(ei_hbm.at[0:2, pl.ds((base + jn) * CHUNK, CHUNK)],
                             pair_v.at[bn], gsem.at[bn])

        return carry

    lax.fori_loop(0, n_w, body, 0)

    def drain(jj, carry):
        b = lax.rem(jj, NBUF)
        pltpu.make_async_copy(ones_v, acc_sh.at[pair_v.at[b, 1]],
                              ssem.at[b]).wait()
        return carry

    lax.fori_loop(n_w - NBUF, n_w, drain, 0)
    plsc.subcore_barrier()

    @pl.when(sid == 0)
    def _():
        pltpu.sync_copy(acc_sh, stage_v)
        pltpu.sync_copy(stage_v, out_hbm.at[pl.ds(cid * NPAD, NPAD)])


@functools.partial(
    pl.kernel,
    out_type=jax.ShapeDtypeStruct((NC, NPAD, D_HID), jnp.float32),
    mesh=_mesh(),
    scratch_types=[
        pltpu.VMEM((CPT, CHUNK), jnp.int32),
        pltpu.VMEM((CPT, CHUNK), jnp.int32),
        pltpu.VMEM((NBUF, CHUNK, D_HID), jnp.float32),
        pltpu.VMEM((NPT, D_HID), jnp.float32),
        pltpu.VMEM_SHARED((NPAD, D_HID), jnp.float32),
        pltpu.SemaphoreType.DMA((NBUF,)),
        pltpu.SemaphoreType.DMA((NBUF,)),
    ],
    compiler_params=pltpu.CompilerParams(use_tc_tiling_on_sc=False),
)
def _agg_kernel(g_hbm, src_hbm, dst_hbm, zero_hbm, out_hbm,
                src_v, dst_v, rows_v, stage_v, acc_sh, gsem, ssem):
    cid = lax.axis_index("c")
    sid = lax.axis_index("s")
    wid = sid * NC + cid
    pltpu.sync_copy(zero_hbm, stage_v)
    pltpu.sync_copy(stage_v, acc_sh.at[pl.ds(sid * NPT, NPT)])
    pltpu.sync_copy(src_hbm.at[pl.ds(wid * CPT, CPT)], src_v)
    pltpu.sync_copy(dst_hbm.at[pl.ds(wid * CPT, CPT)], dst_v)
    plsc.subcore_barrier()

    for k in range(NBUF - 1):
        pltpu.async_copy(g_hbm.at[src_v.at[k]], rows_v.at[k], gsem.at[k])

    def body(j, carry):
        b = lax.rem(j, NBUF)
        pltpu.make_async_copy(g_hbm.at[src_v.at[j]], rows_v.at[b],
                              gsem.at[b]).wait()
        pltpu.async_copy(rows_v.at[b], acc_sh.at[dst_v.at[j]], ssem.at[b],
                         add=True)
        jn = j + NBUF - 1

        @pl.when(jn < CPT)
        def _():
            bn = lax.rem(jn, NBUF)

            @pl.when(j >= 1)
            def _():
                pltpu.make_async_copy(rows_v.at[bn], acc_sh.at[dst_v.at[j - 1]],
                                      ssem.at[bn]).wait()

            pltpu.async_copy(g_hbm.at[src_v.at[jn]], rows_v.at[bn], gsem.at[bn])

        return carry

    lax.fori_loop(0, CPT, body, 0)

    def drain(jj, carry):
        b = lax.rem(jj, NBUF)
        pltpu.make_async_copy(rows_v.at[b], acc_sh.at[dst_v.at[jj]],
                              ssem.at[b]).wait()
        return carry

    lax.fori_loop(CPT - NBUF, CPT, drain, 0)
    plsc.subcore_barrier()
    pltpu.sync_copy(acc_sh.at[pl.ds(sid * NPT, NPT)], stage_v)
    pltpu.sync_copy(stage_v, out_hbm.at[cid, pl.ds(sid * NPT, NPT)])


def _mmh_body(x_ref, w_ref, h_ref):
    h_ref[...] = jnp.dot(x_ref[...], w_ref[...],
                         preferred_element_type=jnp.float32)


def _mm2_body(s1_ref, w2_ref, h2_ref):
    t = jnp.maximum(s1_ref[...], 0.0)
    h2_ref[...] = jnp.dot(t, w2_ref[...], preferred_element_type=jnp.float32)


_F32 = jnp.float32


def kernel(x, edge_index, W1, b1, W2, b2):
    ei32 = edge_index.astype(jnp.int32)
    src = ei32[0]
    dst = ei32[1]
    pad_e = EPAD - N_EDGES
    pad_idx = NPAD - CHUNK + jnp.tile(jnp.arange(CHUNK, dtype=jnp.int32),
                                      pad_e // CHUNK)
    src2 = jnp.concatenate([src, pad_idx]).reshape(NW * CPT, CHUNK)
    dst2 = jnp.concatenate([dst, pad_idx]).reshape(NW * CPT, CHUNK)
    W2p = jnp.pad(W2, ((0, 0), (0, D_HID - N_CLASSES)))
    b1r = jnp.reshape(b1, (1, D_HID))
    b2r = jnp.pad(b2, (0, D_HID - N_CLASSES)).reshape(1, D_HID)

    ones_c = jnp.ones((CHUNK,), _F32)
    zero_n = jnp.zeros((NPAD,), _F32)
    zero_nd = jnp.zeros((NPT, D_HID), _F32)

    degp = _deg_kernel(ei32, ones_c, zero_n).reshape(NC, NPAD)

    h1 = pl.pallas_call(
        _mmh_body,
        out_shape=jax.ShapeDtypeStruct((N_NODES, D_HID), _F32),
    )(x, W1)

    dinv = lax.rsqrt(degp[0] + degp[1] + 1.0)
    dcol = dinv[:, None]
    g1 = jnp.pad(h1 * dcol[:N_NODES], ((0, NPAD - N_NODES), (0, 0)))

    a1 = _agg_kernel(g1, src2, dst2, zero_nd)

    s1 = dcol * (a1[0] + a1[1] + g1) + b1r

    h2 = pl.pallas_call(
        _mm2_body,
        out_shape=jax.ShapeDtypeStruct((NPAD, D_HID), _F32),
    )(s1, W2p)
    g2 = h2 * dcol

    a2 = _agg_kernel(g2, src2, dst2, zero_nd)

    z = (dcol * (a2[0] + a2[1] + g2) + b2r)[:N_NODES, :N_CLASSES]
    return 1.0 / (1.0 + jnp.exp(-z))


# speedup vs baseline: 1.0550x; 1.0550x over previous
import functools

import jax
import jax.numpy as jnp
from jax import lax
from jax.experimental import pallas as pl
from jax.experimental.pallas import tpu as pltpu
from jax.experimental.pallas import tpu_sc as plsc

N_NODES = 10000
N_EDGES = 320000
D_FEAT = 128
D_HID = 16
N_CLASSES = 10

NC = 2
NS = 16
NW = NC * NS
CHUNK = 128
NPAD = 10240
NPT = NPAD // NS
CPT = 80
NBUF = 6
EPAD = NW * CPT * CHUNK


def _mesh():
    return plsc.VectorSubcoreMesh(
        core_axis_name="c", subcore_axis_name="s", num_cores=NC, num_subcores=NS
    )


RCHUNKS = N_EDGES // CHUNK


@functools.partial(
    pl.kernel,
    out_type=jax.ShapeDtypeStruct((NC * NPAD,), jnp.float32),
    mesh=_mesh(),
    scratch_types=[
        pltpu.VMEM((NBUF, 2, CHUNK), jnp.int32),
        pltpu.VMEM((CHUNK,), jnp.float32),
        pltpu.VMEM((NPAD,), jnp.float32),
        pltpu.VMEM_SHARED((NPAD,), jnp.float32),
        pltpu.SemaphoreType.DMA((NBUF,)),
        pltpu.SemaphoreType.DMA((NBUF,)),
    ],
)
def _deg_kernel(ei_hbm, ones_hbm, zero_hbm, out_hbm, pair_v, ones_v, stage_v,
                acc_sh, gsem, ssem):
    cid = lax.axis_index("c")
    sid = lax.axis_index("s")
    wid = sid * NC + cid
    pltpu.sync_copy(ones_hbm, ones_v)

    @pl.when(sid == 0)
    def _():
        pltpu.sync_copy(zero_hbm, stage_v)
        pltpu.sync_copy(stage_v, acc_sh)

    plsc.subcore_barrier()

    base = 78 * wid + jnp.minimum(wid, 4)
    n_w = jnp.where(wid < 4, 79, 78)

    for k in range(NBUF - 1):
        @pl.when(k < n_w)
        def _():
            pltpu.async_copy(ei_hbm.at[0:2, pl.ds((base + k) * CHUNK, CHUNK)],
                             pair_v.at[k], gsem.at[k])

    def body(j, carry):
        b = lax.rem(j, NBUF)
        pltpu.make_async_copy(ei_hbm.at[0:2, pl.ds((base + j) * CHUNK, CHUNK)],
                              pair_v.at[b], gsem.at[b]).wait()
        pltpu.async_copy(ones_v, acc_sh.at[pair_v.at[b, 1]], ssem.at[b],
                         add=True)
        jn = j + NBUF - 1

        @pl.when(jn < n_w)
        def _():
            bn = lax.rem(jn, NBUF)

            @pl.when(j >= 1)
            def _():
                pltpu.make_async_copy(ones_v, acc_sh.at[pair_v.at[bn, 1]],
                                      ssem.at[bn]).wait()

            pltpu.async_copy(ei_hbm.at[0:2, pl.ds((base + jn) * CHUNK, CHUNK)],
                             pair_v.at[bn], gsem.at[bn])

        return carry

    lax.fori_loop(0, n_w, body, 0)

    def drain(jj, carry):
        b = lax.rem(jj, NBUF)
        pltpu.make_async_copy(ones_v, acc_sh.at[pair_v.at[b, 1]],
                              ssem.at[b]).wait()
        return carry

    lax.fori_loop(n_w - NBUF, n_w, drain, 0)
    plsc.subcore_barrier()

    @pl.when(sid == 0)
    def _():
        pltpu.sync_copy(acc_sh, stage_v)
        pltpu.sync_copy(stage_v, out_hbm.at[pl.ds(cid * NPAD, NPAD)])


@functools.partial(
    pl.kernel,
    out_type=jax.ShapeDtypeStruct((NC, NPAD, D_HID), jnp.float32),
    mesh=_mesh(),
    scratch_types=[
        pltpu.VMEM((CPT, CHUNK), jnp.int32),
        pltpu.VMEM((CPT, CHUNK), jnp.int32),
        pltpu.VMEM((NBUF, CHUNK, D_HID), jnp.float32),
        pltpu.VMEM((NPT, D_HID), jnp.float32),
        pltpu.VMEM_SHARED((NPAD, D_HID), jnp.float32),
        pltpu.SemaphoreType.DMA((NBUF,)),
        pltpu.SemaphoreType.DMA((NBUF,)),
    ],
    compiler_params=pltpu.CompilerParams(use_tc_tiling_on_sc=False),
)
def _agg_kernel(g_hbm, src_hbm, dst_hbm, zero_hbm, out_hbm,
                src_v, dst_v, rows_v, stage_v, acc_sh, gsem, ssem):
    cid = lax.axis_index("c")
    sid = lax.axis_index("s")
    wid = sid * NC + cid
    pltpu.sync_copy(zero_hbm, stage_v)
    pltpu.sync_copy(stage_v, acc_sh.at[pl.ds(sid * NPT, NPT)])
    pltpu.sync_copy(src_hbm.at[pl.ds(wid * CPT, CPT)], src_v)
    pltpu.sync_copy(dst_hbm.at[pl.ds(wid * CPT, CPT)], dst_v)
    plsc.subcore_barrier()

    for k in range(NBUF - 1):
        pltpu.async_copy(g_hbm.at[src_v.at[k]], rows_v.at[k], gsem.at[k])

    def body(j, carry):
        b = lax.rem(j, NBUF)
        pltpu.make_async_copy(g_hbm.at[src_v.at[j]], rows_v.at[b],
                              gsem.at[b]).wait()
        pltpu.async_copy(rows_v.at[b], acc_sh.at[dst_v.at[j]], ssem.at[b],
                         add=True)
        jn = j + NBUF - 1

        @pl.when(jn < CPT)
        def _():
            bn = lax.rem(jn, NBUF)

            @pl.when(j >= 1)
            def _():
                pltpu.make_async_copy(rows_v.at[bn], acc_sh.at[dst_v.at[j - 1]],
                                      ssem.at[bn]).wait()

            pltpu.async_copy(g_hbm.at[src_v.at[jn]], rows_v.at[bn], gsem.at[bn])

        return carry

    lax.fori_loop(0, CPT, body, 0)

    def drain(jj, carry):
        b = lax.rem(jj, NBUF)
        pltpu.make_async_copy(rows_v.at[b], acc_sh.at[dst_v.at[jj]],
                              ssem.at[b]).wait()
        return carry

    lax.fori_loop(CPT - NBUF, CPT, drain, 0)
    plsc.subcore_barrier()
    pltpu.sync_copy(acc_sh.at[pl.ds(sid * NPT, NPT)], stage_v)
    pltpu.sync_copy(stage_v, out_hbm.at[cid, pl.ds(sid * NPT, NPT)])


def _mm1_body(degp_ref, x_ref, w_ref, g_ref, dinv_ref):
    deg_row = degp_ref[0:1, :] + degp_ref[1:2, :] + 1.0
    dinv_row = lax.rsqrt(deg_row)
    dinv = lax.dot_general(dinv_row, jnp.ones((1, D_HID), jnp.float32),
                           (((0,), (0,)), ((), ())),
                           preferred_element_type=jnp.float32)
    dinv_ref[...] = dinv
    h = jnp.dot(x_ref[...], w_ref[...], preferred_element_type=jnp.float32)
    g_ref[0:N_NODES, :] = h * dinv[0:N_NODES]
    g_ref[N_NODES:NPAD, :] = jnp.zeros((NPAD - N_NODES, D_HID), jnp.float32)


def _mm2_body(s1_ref, dinv_ref, w2_ref, g2_ref):
    t = jnp.maximum(s1_ref[...], 0.0)
    h2 = jnp.dot(t, w2_ref[...], preferred_element_type=jnp.float32)
    g2_ref[...] = h2 * dinv_ref[...]


_F32 = jnp.float32


def kernel(x, edge_index, W1, b1, W2, b2):
    ei32 = edge_index.astype(jnp.int32)
    src = ei32[0]
    dst = ei32[1]
    pad_e = EPAD - N_EDGES
    pad_idx = NPAD - CHUNK + jnp.tile(jnp.arange(CHUNK, dtype=jnp.int32),
                                      pad_e // CHUNK)
    src2 = jnp.concatenate([src, pad_idx]).reshape(NW * CPT, CHUNK)
    dst2 = jnp.concatenate([dst, pad_idx]).reshape(NW * CPT, CHUNK)
    W2p = jnp.pad(W2, ((0, 0), (0, D_HID - N_CLASSES)))
    b1r = jnp.reshape(b1, (1, D_HID))
    b2r = jnp.pad(b2, (0, D_HID - N_CLASSES)).reshape(1, D_HID)

    ones_c = jnp.ones((CHUNK,), _F32)
    zero_n = jnp.zeros((NPAD,), _F32)
    zero_nd = jnp.zeros((NPT, D_HID), _F32)

    degp = _deg_kernel(ei32, ones_c, zero_n).reshape(NC, NPAD)

    g1, dinv16 = pl.pallas_call(
        _mm1_body,
        out_shape=[jax.ShapeDtypeStruct((NPAD, D_HID), _F32),
                   jax.ShapeDtypeStruct((NPAD, D_HID), _F32)],
    )(degp, x, W1)

    a1 = _agg_kernel(g1, src2, dst2, zero_nd)

    s1 = dinv16 * (a1[0] + a1[1] + g1) + b1r

    g2 = pl.pallas_call(
        _mm2_body,
        out_shape=jax.ShapeDtypeStruct((NPAD, D_HID), _F32),
    )(s1, dinv16, W2p)

    a2 = _agg_kernel(g2, src2, dst2, zero_nd)

    z = (dinv16 * (a2[0] + a2[1] + g2) + b2r)[:N_NODES, :N_CLASSES]
    return 1.0 / (1.0 + jnp.exp(-z))


# speedup vs baseline: 1.0882x; 1.0315x over previous
import functools

import jax
import jax.numpy as jnp
from jax import lax
from jax.experimental import pallas as pl
from jax.experimental.pallas import tpu as pltpu
from jax.experimental.pallas import tpu_sc as plsc

N_NODES = 10000
N_EDGES = 320000
D_FEAT = 128
D_HID = 16
N_CLASSES = 10

NC = 2
NS = 16
NW = NC * NS
CHUNK = 128
NPAD = 10240
NPT = NPAD // NS
CPT = 80
NBUF = 8
EPAD = NW * CPT * CHUNK


def _mesh():
    return plsc.VectorSubcoreMesh(
        core_axis_name="c", subcore_axis_name="s", num_cores=NC, num_subcores=NS
    )


RCHUNKS = N_EDGES // CHUNK


@functools.partial(
    pl.kernel,
    out_type=jax.ShapeDtypeStruct((NC * NPAD,), jnp.float32),
    mesh=_mesh(),
    scratch_types=[
        pltpu.VMEM((NBUF, 2, CHUNK), jnp.int32),
        pltpu.VMEM((CHUNK,), jnp.float32),
        pltpu.VMEM((NPAD,), jnp.float32),
        pltpu.VMEM_SHARED((NPAD,), jnp.float32),
        pltpu.SemaphoreType.DMA((NBUF,)),
        pltpu.SemaphoreType.DMA((NBUF,)),
    ],
)
def _deg_kernel(ei_hbm, ones_hbm, zero_hbm, out_hbm, pair_v, ones_v, stage_v,
                acc_sh, gsem, ssem):
    cid = lax.axis_index("c")
    sid = lax.axis_index("s")
    wid = sid * NC + cid
    pltpu.sync_copy(ones_hbm, ones_v)

    @pl.when(sid == 0)
    def _():
        pltpu.sync_copy(zero_hbm, stage_v)
        pltpu.sync_copy(stage_v, acc_sh)

    plsc.subcore_barrier()

    base = 78 * wid + jnp.minimum(wid, 4)
    n_w = jnp.where(wid < 4, 79, 78)

    for k in range(NBUF - 1):
        @pl.when(k < n_w)
        def _():
            pltpu.async_copy(ei_hbm.at[0:2, pl.ds((base + k) * CHUNK, CHUNK)],
                             pair_v.at[k], gsem.at[k])

    def body(j, carry):
        b = lax.rem(j, NBUF)
        pltpu.make_async_copy(ei_hbm.at[0:2, pl.ds((base + j) * CHUNK, CHUNK)],
                              pair_v.at[b], gsem.at[b]).wait()
        pltpu.async_copy(ones_v, acc_sh.at[pair_v.at[b, 1]], ssem.at[b],
                         add=True)
        jn = j + NBUF - 1

        @pl.when(jn < n_w)
        def _():
            bn = lax.rem(jn, NBUF)

            @pl.when(j >= 1)
            def _():
                pltpu.make_async_copy(ones_v, acc_sh.at[pair_v.at[bn, 1]],
                                      ssem.at[bn]).wait()

            pltpu.async_copy(ei_hbm.at[0:2, pl.ds((base + jn) * CHUNK, CHUNK)],
                             pair_v.at[bn], gsem.at[bn])

        return carry

    lax.fori_loop(0, n_w, body, 0)

    def drain(jj, carry):
        b = lax.rem(jj, NBUF)
        pltpu.make_async_copy(ones_v, acc_sh.at[pair_v.at[b, 1]],
                              ssem.at[b]).wait()
        return carry

    lax.fori_loop(n_w - NBUF, n_w, drain, 0)
    plsc.subcore_barrier()

    @pl.when(sid == 0)
    def _():
        pltpu.sync_copy(acc_sh, stage_v)
        pltpu.sync_copy(stage_v, out_hbm.at[pl.ds(cid * NPAD, NPAD)])


@functools.partial(
    pl.kernel,
    out_type=jax.ShapeDtypeStruct((NC, NPAD, D_HID), jnp.float32),
    mesh=_mesh(),
    scratch_types=[
        pltpu.VMEM((CPT, CHUNK), jnp.int32),
        pltpu.VMEM((CPT, CHUNK), jnp.int32),
        pltpu.VMEM((NBUF, CHUNK, D_HID), jnp.float32),
        pltpu.VMEM((NPT, D_HID), jnp.float32),
        pltpu.VMEM_SHARED((NPAD, D_HID), jnp.float32),
        pltpu.SemaphoreType.DMA((NBUF,)),
        pltpu.SemaphoreType.DMA((NBUF,)),
    ],
    compiler_params=pltpu.CompilerParams(use_tc_tiling_on_sc=False),
)
def _agg_kernel(g_hbm, src_hbm, dst_hbm, zero_hbm, out_hbm,
                src_v, dst_v, rows_v, stage_v, acc_sh, gsem, ssem):
    cid = lax.axis_index("c")
    sid = lax.axis_index("s")
    wid = sid * NC + cid
    pltpu.sync_copy(zero_hbm, stage_v)
    pltpu.sync_copy(stage_v, acc_sh.at[pl.ds(sid * NPT, NPT)])
    pltpu.sync_copy(src_hbm.at[pl.ds(wid * CPT, CPT)], src_v)
    pltpu.sync_copy(dst_hbm.at[pl.ds(wid * CPT, CPT)], dst_v)
    plsc.subcore_barrier()

    for k in range(NBUF - 1):
        pltpu.async_copy(g_hbm.at[src_v.at[k]], rows_v.at[k], gsem.at[k])

    def body(j, carry):
        b = lax.rem(j, NBUF)
        pltpu.make_async_copy(g_hbm.at[src_v.at[j]], rows_v.at[b],
                              gsem.at[b]).wait()
        pltpu.async_copy(rows_v.at[b], acc_sh.at[dst_v.at[j]], ssem.at[b],
                         add=True)
        jn = j + NBUF - 1

        @pl.when(jn < CPT)
        def _():
            bn = lax.rem(jn, NBUF)

            @pl.when(j >= 1)
            def _():
                pltpu.make_async_copy(rows_v.at[bn], acc_sh.at[dst_v.at[j - 1]],
                                      ssem.at[bn]).wait()

            pltpu.async_copy(g_hbm.at[src_v.at[jn]], rows_v.at[bn], gsem.at[bn])

        return carry

    lax.fori_loop(0, CPT, body, 0)

    def drain(jj, carry):
        b = lax.rem(jj, NBUF)
        pltpu.make_async_copy(rows_v.at[b], acc_sh.at[dst_v.at[jj]],
                              ssem.at[b]).wait()
        return carry

    lax.fori_loop(CPT - NBUF, CPT, drain, 0)
    plsc.subcore_barrier()
    pltpu.sync_copy(acc_sh.at[pl.ds(sid * NPT, NPT)], stage_v)
    pltpu.sync_copy(stage_v, out_hbm.at[cid, pl.ds(sid * NPT, NPT)])


def _mm1_body(degp_ref, x_ref, w_ref, g_ref, dinv_ref):
    deg_row = degp_ref[0:1, :] + degp_ref[1:2, :] + 1.0
    dinv_row = lax.rsqrt(deg_row)
    dinv = lax.dot_general(dinv_row, jnp.ones((1, D_HID), jnp.float32),
                           (((0,), (0,)), ((), ())),
                           preferred_element_type=jnp.float32)
    dinv_ref[...] = dinv
    h = jnp.dot(x_ref[...], w_ref[...], preferred_element_type=jnp.float32)
    g_ref[0:N_NODES, :] = h * dinv[0:N_NODES]
    g_ref[N_NODES:NPAD, :] = jnp.zeros((NPAD - N_NODES, D_HID), jnp.float32)


def _mm2_body(s1_ref, dinv_ref, w2_ref, g2_ref):
    t = jnp.maximum(s1_ref[...], 0.0)
    h2 = jnp.dot(t, w2_ref[...], preferred_element_type=jnp.float32)
    g2_ref[...] = h2 * dinv_ref[...]


_F32 = jnp.float32


def kernel(x, edge_index, W1, b1, W2, b2):
    ei32 = edge_index.astype(jnp.int32)
    src = ei32[0]
    dst = ei32[1]
    pad_e = EPAD - N_EDGES
    pad_idx = NPAD - CHUNK + jnp.tile(jnp.arange(CHUNK, dtype=jnp.int32),
                                      pad_e // CHUNK)
    src2 = jnp.concatenate([src, pad_idx]).reshape(NW * CPT, CHUNK)
    dst2 = jnp.concatenate([dst, pad_idx]).reshape(NW * CPT, CHUNK)
    W2p = jnp.pad(W2, ((0, 0), (0, D_HID - N_CLASSES)))
    b1r = jnp.reshape(b1, (1, D_HID))
    b2r = jnp.pad(b2, (0, D_HID - N_CLASSES)).reshape(1, D_HID)

    ones_c = jnp.ones((CHUNK,), _F32)
    zero_n = jnp.zeros((NPAD,), _F32)
    zero_nd = jnp.zeros((NPT, D_HID), _F32)

    degp = _deg_kernel(ei32, ones_c, zero_n).reshape(NC, NPAD)

    g1, dinv16 = pl.pallas_call(
        _mm1_body,
        out_shape=[jax.ShapeDtypeStruct((NPAD, D_HID), _F32),
                   jax.ShapeDtypeStruct((NPAD, D_HID), _F32)],
    )(degp, x, W1)

    a1 = _agg_kernel(g1, src2, dst2, zero_nd)

    s1 = dinv16 * (a1[0] + a1[1] + g1) + b1r

    g2 = pl.pallas_call(
        _mm2_body,
        out_shape=jax.ShapeDtypeStruct((NPAD, D_HID), _F32),
    )(s1, dinv16, W2p)

    a2 = _agg_kernel(g2, src2, dst2, zero_nd)

    z = (dinv16 * (a2[0] + a2[1] + g2) + b2r)[:N_NODES, :N_CLASSES]
    return 1.0 / (1.0 + jnp.exp(-z))
